# Initial kernel scaffold; baseline (speedup 1.0000x reference)
import jax
import jax.numpy as jnp
from jax import lax
from jax.experimental import pallas as pl
from jax.experimental.pallas import tpu as pltpu
from jax.experimental.pallas import tpu_sc as plsc

N0, N1, N2 = 50000, 10240, 1024
E0, E1 = 256000, 25600
D_IN, D_HID, D_OUT = 128, 128, 64

NC, NS, L = 2, 16, 16
NW = NC * NS
E0_W = E0 // NW
E1_W = E1 // NW
K = 128
TRASH = N2
R_ACC = N2 + L
RPT = R_ACC // NS
E1_PAD = (E1_W + K - 1) // K * K
NCH1 = E1_PAD // K

_mesh = plsc.VectorSubcoreMesh(core_axis_name="c", subcore_axis_name="s")


def _sc_agg0(x_hbm, ei0_hbm, ei1_hbm, s0p_hbm, cnt0p_hbm, cnt1p_hbm,
             agg_sh, srcstage, dststage, srcflat, dstflat, cs, cd, rows,
             cnt0loc, d1stage, cnt1loc, sem):
    cid = lax.axis_index("c")
    sid = lax.axis_index("s")
    wid = sid * NC + cid

    z16f = jnp.zeros((L,), jnp.float32)
    ones16 = jnp.ones((L,), jnp.float32)

    def _zrow(i, _):
        for k in range(D_IN // L):
            rows[i, pl.ds(k * L, L)] = z16f
        return 0
    lax.fori_loop(0, K, _zrow, 0)

    def _zc0(i, _):
        cnt0loc[pl.ds(i * L, L)] = z16f
        return 0
    lax.fori_loop(0, R_ACC // L, _zc0, 0)

    def _zc1(i, _):
        cnt1loc[pl.ds(i * L, L)] = z16f
        return 0
    lax.fori_loop(0, N2 // L, _zc1, 0)

    pltpu.sync_copy(rows.at[pl.ds(0, RPT)], agg_sh.at[pl.ds(sid * RPT, RPT)])
    plsc.subcore_barrier()

    base = wid * E0_W
    pltpu.sync_copy(ei0_hbm.at[0, pl.ds(base, E0_W)], srcstage)
    pltpu.sync_copy(ei0_hbm.at[1, pl.ds(base, E0_W)], dststage)

    def _filt(i, off):
        s16 = srcstage[pl.ds(i * L, L)]
        d16 = dststage[pl.ds(i * L, L)]
        mask = d16 < N2
        plsc.store_compressed(srcflat.at[pl.ds(off, L)], s16, mask=mask)
        plsc.store_compressed(dstflat.at[pl.ds(off, L)], d16, mask=mask)
        dcl = jnp.where(mask, d16, TRASH)
        plsc.addupdate_scatter(cnt0loc, [dcl], ones16)
        return off + jnp.sum(mask.astype(jnp.int32))
    n = lax.fori_loop(0, E0_W // L, _filt, jnp.int32(0))

    z16i = jnp.zeros((L,), jnp.int32)
    t16i = jnp.full((L,), TRASH, jnp.int32)
    for t in range(K // L):
        srcflat[pl.ds(n + t * L, L)] = z16i
        dstflat[pl.ds(n + t * L, L)] = t16i

    def _chunk(j, _):
        cb = j * K
        for k in range(K // L):
            cs[pl.ds(k * L, L)] = srcflat[pl.ds(cb + k * L, L)]
            cd[pl.ds(k * L, L)] = dstflat[pl.ds(cb + k * L, L)]
        pltpu.async_copy(x_hbm.at[cs], rows, sem).wait()
        pltpu.sync_copy(rows, agg_sh.at[cd], add=True)
        return 0
    nch = (n + (K - 1)) // K
    lax.fori_loop(0, nch, _chunk, 0)

    base1 = wid * E1_W
    pltpu.sync_copy(ei1_hbm.at[1, pl.ds(base1, E1_W)], d1stage)

    def _c1(i, _):
        d16 = d1stage[pl.ds(i * L, L)]
        plsc.addupdate_scatter(cnt1loc, [d16], ones16)
        return 0
    lax.fori_loop(0, E1_W // L, _c1, 0)

    plsc.subcore_barrier()

    pltpu.sync_copy(agg_sh.at[pl.ds(sid * RPT, RPT)], rows.at[pl.ds(0, RPT)])
    pltpu.sync_copy(rows.at[pl.ds(0, RPT)],
                    s0p_hbm.at[cid, pl.ds(sid * RPT, RPT)])
    pltpu.sync_copy(cnt0loc, cnt0p_hbm.at[wid])
    pltpu.sync_copy(cnt1loc, cnt1p_hbm.at[wid])


_agg0_call = pl.kernel(
    _sc_agg0,
    out_type=[
        jax.ShapeDtypeStruct((NC, R_ACC, D_IN), jnp.float32),
        jax.ShapeDtypeStruct((NW, R_ACC), jnp.float32),
        jax.ShapeDtypeStruct((NW, N2), jnp.float32),
    ],
    mesh=_mesh,
    scratch_types=[
        pltpu.VMEM_SHARED((R_ACC, D_IN), jnp.float32),
        pltpu.VMEM((E0_W,), jnp.int32),
        pltpu.VMEM((E0_W,), jnp.int32),
        pltpu.VMEM((E0_W + 2 * K,), jnp.int32),
        pltpu.VMEM((E0_W + 2 * K,), jnp.int32),
        pltpu.VMEM((K,), jnp.int32),
        pltpu.VMEM((K,), jnp.int32),
        pltpu.VMEM((K, D_IN), jnp.float32),
        pltpu.VMEM((R_ACC,), jnp.float32),
        pltpu.VMEM((E1_W,), jnp.int32),
        pltpu.VMEM((N2,), jnp.float32),
        pltpu.SemaphoreType.DMA,
    ],
)


def _sc_agg1(m1_hbm, ei1_hbm, s1p_hbm, agg_sh, sstage, dstage, cs, cd, rows,
             sem):
    cid = lax.axis_index("c")
    sid = lax.axis_index("s")
    wid = sid * NC + cid

    z16f = jnp.zeros((L,), jnp.float32)

    def _zrow(i, _):
        for k in range(D_OUT // L):
            rows[i, pl.ds(k * L, L)] = z16f
        return 0
    lax.fori_loop(0, K, _zrow, 0)
    pltpu.sync_copy(rows.at[pl.ds(0, RPT)], agg_sh.at[pl.ds(sid * RPT, RPT)])
    plsc.subcore_barrier()

    base = wid * E1_W
    pltpu.sync_copy(ei1_hbm.at[0, pl.ds(base, E1_W)], sstage.at[pl.ds(0, E1_W)])
    pltpu.sync_copy(ei1_hbm.at[1, pl.ds(base, E1_W)], dstage.at[pl.ds(0, E1_W)])
    z16i = jnp.zeros((L,), jnp.int32)
    t16i = jnp.full((L,), TRASH, jnp.int32)
    for t in range((E1_PAD - E1_W) // L):
        sstage[pl.ds(E1_W + t * L, L)] = z16i
        dstage[pl.ds(E1_W + t * L, L)] = t16i

    for j in range(NCH1):
        for k in range(K // L):
            cs[pl.ds(k * L, L)] = sstage[pl.ds(j * K + k * L, L)]
            cd[pl.ds(k * L, L)] = dstage[pl.ds(j * K + k * L, L)]
        pltpu.async_copy(m1_hbm.at[cs], rows, sem).wait()
        pltpu.sync_copy(rows, agg_sh.at[cd], add=True)

    plsc.subcore_barrier()
    pltpu.sync_copy(agg_sh.at[pl.ds(sid * RPT, RPT)], rows.at[pl.ds(0, RPT)])
    pltpu.sync_copy(rows.at[pl.ds(0, RPT)],
                    s1p_hbm.at[cid, pl.ds(sid * RPT, RPT)])


_agg1_call = pl.kernel(
    _sc_agg1,
    out_type=jax.ShapeDtypeStruct((NC, R_ACC, D_OUT), jnp.float32),
    mesh=_mesh,
    scratch_types=[
        pltpu.VMEM_SHARED((R_ACC, D_OUT), jnp.float32),
        pltpu.VMEM((E1_PAD,), jnp.int32),
        pltpu.VMEM((E1_PAD,), jnp.int32),
        pltpu.VMEM((K,), jnp.int32),
        pltpu.VMEM((K,), jnp.int32),
        pltpu.VMEM((K, D_OUT), jnp.float32),
        pltpu.SemaphoreType.DMA,
    ],
)


def _tc_mid(s0p_ref, cnt0p_ref, x1_ref, w0l_ref, b0_ref, w0r_ref, w1l_ref,
            w1r_ref, b1_ref, m1_ref, r1_ref):
    s0 = s0p_ref[0, :N2, :] + s0p_ref[1, :N2, :]
    cnt = jnp.sum(cnt0p_ref[:, :N2], axis=0)
    agg = s0 / jnp.clip(cnt, 1.0)[:, None]
    h = (jnp.dot(agg, w0l_ref[...], preferred_element_type=jnp.float32)
         + b0_ref[0, :][None, :]
         + jnp.dot(x1_ref[...], w0r_ref[...],
                   preferred_element_type=jnp.float32))
    h = jnp.maximum(h, 0.0)
    m1_ref[...] = jnp.dot(h, w1l_ref[...], preferred_element_type=jnp.float32)
    r1_ref[...] = (jnp.dot(h, w1r_ref[...], preferred_element_type=jnp.float32)
                   + b1_ref[0, :][None, :])


def _tc_out(s1p_ref, cnt1p_ref, r1_ref, out_ref):
    s1 = s1p_ref[0, :N2, :] + s1p_ref[1, :N2, :]
    cnt = jnp.sum(cnt1p_ref[...], axis=0)
    z = s1 / jnp.clip(cnt, 1.0)[:, None] + r1_ref[...]
    m = jnp.max(z, axis=-1, keepdims=True)
    lse = jnp.log(jnp.sum(jnp.exp(z - m), axis=-1, keepdims=True)) + m
    out_ref[...] = z - lse


@jax.jit
def kernel(x, edge_index0, edge_index1, W0l, b0, W0r, W1l, b1, W1r):
    ei0 = edge_index0.astype(jnp.int32)
    ei1 = edge_index1.astype(jnp.int32)

    s0p, cnt0p, cnt1p = _agg0_call(x, ei0, ei1)

    m1, r1 = pl.pallas_call(
        _tc_mid,
        out_shape=[
            jax.ShapeDtypeStruct((N2, D_OUT), jnp.float32),
            jax.ShapeDtypeStruct((N2, D_OUT), jnp.float32),
        ],
    )(s0p, cnt0p, x[:N2], W0l, b0.reshape(1, -1), W0r, W1l, W1r,
      b1.reshape(1, -1))

    s1p = _agg1_call(m1, ei1)

    out = pl.pallas_call(
        _tc_out,
        out_shape=jax.ShapeDtypeStruct((N2, D_OUT), jnp.float32),
    )(s1p, cnt1p, r1)
    return out


# speedup vs baseline: 7.0747x; 7.0747x over previous
import jax
import jax.numpy as jnp
from jax import lax
from jax.experimental import pallas as pl
from jax.experimental.pallas import tpu as pltpu
from jax.experimental.pallas import tpu_sc as plsc

N0, N1, N2 = 50000, 10240, 1024
E0, E1 = 256000, 25600
D_IN, D_HID, D_OUT = 128, 128, 64

NC, NS, L = 2, 16, 16
NW = NC * NS
E0_W = E0 // NW
E1_W = E1 // NW
K = 128
TRASH = N2
R_ACC = 1152
RPT = R_ACC // NS
E1_PAD = (E1_W + K - 1) // K * K
NCH1 = E1_PAD // K

_mesh = plsc.VectorSubcoreMesh(core_axis_name="c", subcore_axis_name="s")


def _sc_agg0(x_hbm, src0_hbm, dst0_hbm, dst1_hbm, s0p_hbm, cnt0p_hbm,
             cnt1p_hbm,
             agg_sh, srcstage, dststage, srcflat, dstflat, cs, cd, rows,
             cnt0loc, d1stage, cnt1loc, sem):
    cid = lax.axis_index("c")
    sid = lax.axis_index("s")
    wid = sid * NC + cid

    z16f = jnp.zeros((L,), jnp.float32)
    ones16 = jnp.ones((L,), jnp.float32)

    def _zrow(i, _):
        for k in range(D_IN // L):
            rows[i, pl.ds(k * L, L)] = z16f
        return 0
    lax.fori_loop(0, K, _zrow, 0)

    def _zc0(i, _):
        cnt0loc[pl.ds(i * L, L)] = z16f
        return 0
    lax.fori_loop(0, R_ACC // L, _zc0, 0)

    def _zc1(i, _):
        cnt1loc[pl.ds(i * L, L)] = z16f
        return 0
    lax.fori_loop(0, N2 // L, _zc1, 0)

    pltpu.sync_copy(rows.at[pl.ds(0, RPT)], agg_sh.at[pl.ds(sid * RPT, RPT)])
    plsc.subcore_barrier()

    base = wid * E0_W
    pltpu.sync_copy(src0_hbm.at[pl.ds(base, E0_W)], srcstage)
    pltpu.sync_copy(dst0_hbm.at[pl.ds(base, E0_W)], dststage)

    def _filt(i, off):
        s16 = srcstage[pl.ds(i * L, L)]
        d16 = dststage[pl.ds(i * L, L)]
        mask = d16 < N2
        mi = mask.astype(jnp.int32)
        pos = off + plsc.cumsum(mi) - 1
        plsc.store_scatter(srcflat, [pos], s16, mask=mask)
        plsc.store_scatter(dstflat, [pos], d16, mask=mask)
        dcl = jnp.where(mask, d16, jnp.full((L,), TRASH, jnp.int32))
        plsc.addupdate_scatter(cnt0loc, [dcl], ones16)
        return off + jnp.sum(mi)
    n = lax.fori_loop(0, E0_W // L, _filt, jnp.int32(0))

    z16i = jnp.zeros((L,), jnp.int32)
    t16i = jnp.full((L,), TRASH, jnp.int32)
    for t in range(K // L):
        srcflat[pl.ds(n + t * L, L)] = z16i
        dstflat[pl.ds(n + t * L, L)] = t16i

    def _chunk(j, _):
        cb = j * K
        for k in range(K // L):
            cs[pl.ds(k * L, L)] = srcflat[pl.ds(cb + k * L, L)]
            cd[pl.ds(k * L, L)] = dstflat[pl.ds(cb + k * L, L)]
        pltpu.async_copy(x_hbm.at[cs], rows, sem).wait()
        pltpu.sync_copy(rows, agg_sh.at[cd], add=True)
        return 0
    nch = (n + (K - 1)) // K
    lax.fori_loop(0, nch, _chunk, 0)

    base1 = wid * E1_W
    pltpu.sync_copy(dst1_hbm.at[pl.ds(base1, E1_W)], d1stage)

    def _c1(i, _):
        d16 = d1stage[pl.ds(i * L, L)]
        plsc.addupdate_scatter(cnt1loc, [d16], ones16)
        return 0
    lax.fori_loop(0, E1_W // L, _c1, 0)

    plsc.subcore_barrier()

    pltpu.sync_copy(agg_sh.at[pl.ds(sid * RPT, RPT)], rows.at[pl.ds(0, RPT)])
    pltpu.sync_copy(rows.at[pl.ds(0, RPT)],
                    s0p_hbm.at[cid, pl.ds(sid * RPT, RPT)])
    pltpu.sync_copy(cnt0loc, cnt0p_hbm.at[pl.ds(wid * R_ACC, R_ACC)])
    pltpu.sync_copy(cnt1loc, cnt1p_hbm.at[pl.ds(wid * N2, N2)])


_agg0_call = pl.kernel(
    _sc_agg0,
    out_type=[
        jax.ShapeDtypeStruct((NC, R_ACC, D_IN), jnp.float32),
        jax.ShapeDtypeStruct((NW * R_ACC,), jnp.float32),
        jax.ShapeDtypeStruct((NW * N2,), jnp.float32),
    ],
    mesh=_mesh,
    scratch_types=[
        pltpu.VMEM_SHARED((R_ACC, D_IN), jnp.float32),
        pltpu.VMEM((E0_W,), jnp.int32),
        pltpu.VMEM((E0_W,), jnp.int32),
        pltpu.VMEM((E0_W + 2 * K,), jnp.int32),
        pltpu.VMEM((E0_W + 2 * K,), jnp.int32),
        pltpu.VMEM((K,), jnp.int32),
        pltpu.VMEM((K,), jnp.int32),
        pltpu.VMEM((K, D_IN), jnp.float32),
        pltpu.VMEM((R_ACC,), jnp.float32),
        pltpu.VMEM((E1_W,), jnp.int32),
        pltpu.VMEM((N2,), jnp.float32),
        pltpu.SemaphoreType.DMA,
    ],
    compiler_params=pltpu.CompilerParams(needs_layout_passes=False),
)


def _sc_agg1(h_hbm, src1_hbm, dst1_hbm, s1p_hbm, agg_sh, sstage, dstage, cs,
             cd, rows, sem):
    cid = lax.axis_index("c")
    sid = lax.axis_index("s")
    wid = sid * NC + cid

    z16f = jnp.zeros((L,), jnp.float32)

    def _zrow(i, _):
        for k in range(D_HID // L):
            rows[i, pl.ds(k * L, L)] = z16f
        return 0
    lax.fori_loop(0, K, _zrow, 0)
    pltpu.sync_copy(rows.at[pl.ds(0, RPT)], agg_sh.at[pl.ds(sid * RPT, RPT)])
    plsc.subcore_barrier()

    base = wid * E1_W
    pltpu.sync_copy(src1_hbm.at[pl.ds(base, E1_W)], sstage.at[pl.ds(0, E1_W)])
    pltpu.sync_copy(dst1_hbm.at[pl.ds(base, E1_W)], dstage.at[pl.ds(0, E1_W)])
    z16i = jnp.zeros((L,), jnp.int32)
    t16i = jnp.full((L,), TRASH, jnp.int32)
    for t in range((E1_PAD - E1_W) // L):
        sstage[pl.ds(E1_W + t * L, L)] = z16i
        dstage[pl.ds(E1_W + t * L, L)] = t16i

    for j in range(NCH1):
        for k in range(K // L):
            cs[pl.ds(k * L, L)] = sstage[pl.ds(j * K + k * L, L)]
            cd[pl.ds(k * L, L)] = dstage[pl.ds(j * K + k * L, L)]
        pltpu.async_copy(h_hbm.at[cs], rows, sem).wait()
        pltpu.sync_copy(rows, agg_sh.at[cd], add=True)

    plsc.subcore_barrier()
    pltpu.sync_copy(agg_sh.at[pl.ds(sid * RPT, RPT)], rows.at[pl.ds(0, RPT)])
    pltpu.sync_copy(rows.at[pl.ds(0, RPT)],
                    s1p_hbm.at[cid, pl.ds(sid * RPT, RPT)])


_agg1_call = pl.kernel(
    _sc_agg1,
    out_type=jax.ShapeDtypeStruct((NC, R_ACC, D_HID), jnp.float32),
    mesh=_mesh,
    scratch_types=[
        pltpu.VMEM_SHARED((R_ACC, D_HID), jnp.float32),
        pltpu.VMEM((E1_PAD,), jnp.int32),
        pltpu.VMEM((E1_PAD,), jnp.int32),
        pltpu.VMEM((K,), jnp.int32),
        pltpu.VMEM((K,), jnp.int32),
        pltpu.VMEM((K, D_HID), jnp.float32),
        pltpu.SemaphoreType.DMA,
    ],
    compiler_params=pltpu.CompilerParams(needs_layout_passes=False),
)


def _tc_mid(s0p_ref, cnt0p_ref, x1_ref, w0l_ref, b0_ref, w0r_ref, w1r_ref,
            b1_ref, h_ref, r1_ref):
    s0 = s0p_ref[0, :N2, :] + s0p_ref[1, :N2, :]
    cnt = jnp.sum(cnt0p_ref[:, :N2], axis=0)
    agg = s0 / jnp.clip(cnt, 1.0)[:, None]
    h = (jnp.dot(agg, w0l_ref[...], preferred_element_type=jnp.float32)
         + b0_ref[0, :][None, :]
         + jnp.dot(x1_ref[...], w0r_ref[...],
                   preferred_element_type=jnp.float32))
    h = jnp.maximum(h, 0.0)
    h_ref[...] = h
    r1_ref[...] = (jnp.dot(h, w1r_ref[...], preferred_element_type=jnp.float32)
                   + b1_ref[0, :][None, :])


def _tc_out(s1p_ref, cnt1p_ref, r1_ref, w1l_ref, out_ref):
    s1 = s1p_ref[0, :N2, :] + s1p_ref[1, :N2, :]
    cnt = jnp.sum(cnt1p_ref[...], axis=0)
    agg = s1 / jnp.clip(cnt, 1.0)[:, None]
    z = (jnp.dot(agg, w1l_ref[...], preferred_element_type=jnp.float32)
         + r1_ref[...])
    m = jnp.max(z, axis=-1, keepdims=True)
    lse = jnp.log(jnp.sum(jnp.exp(z - m), axis=-1, keepdims=True)) + m
    out_ref[...] = z - lse


@jax.jit
def kernel(x, edge_index0, edge_index1, W0l, b0, W0r, W1l, b1, W1r):
    ei0 = edge_index0.astype(jnp.int32)
    ei1 = edge_index1.astype(jnp.int32)
    src0, dst0 = ei0[0], ei0[1]
    src1, dst1 = ei1[0], ei1[1]

    s0p, cnt0p, cnt1p = _agg0_call(x, src0, dst0, dst1)
    cnt0p = cnt0p.reshape(NW, R_ACC)
    cnt1p = cnt1p.reshape(NW, N2)

    h, r1 = pl.pallas_call(
        _tc_mid,
        out_shape=[
            jax.ShapeDtypeStruct((N2, D_HID), jnp.float32),
            jax.ShapeDtypeStruct((N2, D_OUT), jnp.float32),
        ],
    )(s0p, cnt0p, x[:N2], W0l, b0.reshape(1, -1), W0r, W1r,
      b1.reshape(1, -1))

    s1p = _agg1_call(h, src1, dst1)

    out = pl.pallas_call(
        _tc_out,
        out_shape=jax.ShapeDtypeStruct((N2, D_OUT), jnp.float32),
    )(s1p, cnt1p, r1, W1l)
    return out


# speedup vs baseline: 7.1474x; 1.0103x over previous
import jax
import jax.numpy as jnp
from jax import lax
from jax.experimental import pallas as pl
from jax.experimental.pallas import tpu as pltpu
from jax.experimental.pallas import tpu_sc as plsc

N0, N1, N2 = 50000, 10240, 1024
E0, E1 = 256000, 25600
D_IN, D_HID, D_OUT = 128, 128, 64

NC, NS, L = 2, 16, 16
NW = NC * NS
E0_W = E0 // NW
E1_W = E1 // NW
K = 128
TRASH = N2
R_ACC = 1152
RPT = R_ACC // NS
E1_PAD = (E1_W + K - 1) // K * K
NCH1 = E1_PAD // K
G = 4

_mesh = plsc.VectorSubcoreMesh(core_axis_name="c", subcore_axis_name="s")


def _sc_agg0(x_hbm, src0_hbm, dst0_hbm, dst1_hbm, s0p_hbm, cnt0p_hbm,
             cnt1p_hbm,
             agg_sh, srcstage, dststage, srcflat, dstflat, cs, cd, rows,
             cnt0loc, d1stage, cnt1loc, sem, ssem):
    cid = lax.axis_index("c")
    sid = lax.axis_index("s")
    wid = sid * NC + cid

    z16f = jnp.zeros((L,), jnp.float32)
    ones16 = jnp.ones((L,), jnp.float32)

    def _zrow(i, _):
        for k in range(D_IN // L):
            rows[i, pl.ds(k * L, L)] = z16f
        return 0
    lax.fori_loop(0, RPT, _zrow, 0)

    def _zc0(i, _):
        cnt0loc[pl.ds(i * L, L)] = z16f
        return 0
    lax.fori_loop(0, R_ACC // L, _zc0, 0)

    def _zc1(i, _):
        cnt1loc[pl.ds(i * L, L)] = z16f
        return 0
    lax.fori_loop(0, N2 // L, _zc1, 0)

    pltpu.sync_copy(rows.at[pl.ds(0, RPT)], agg_sh.at[pl.ds(sid * RPT, RPT)])
    plsc.subcore_barrier()

    base = wid * E0_W
    pltpu.sync_copy(src0_hbm.at[pl.ds(base, E0_W)], srcstage)
    pltpu.sync_copy(dst0_hbm.at[pl.ds(base, E0_W)], dststage)

    def _filt(i, off):
        s16 = srcstage[pl.ds(i * L, L)]
        d16 = dststage[pl.ds(i * L, L)]
        mask = d16 < N2
        mi = mask.astype(jnp.int32)
        pos = off + plsc.cumsum(mi) - 1
        plsc.store_scatter(srcflat, [pos], s16, mask=mask)
        plsc.store_scatter(dstflat, [pos], d16, mask=mask)
        dcl = jnp.where(mask, d16, jnp.full((L,), TRASH, jnp.int32))
        plsc.addupdate_scatter(cnt0loc, [dcl], ones16)
        return off + jnp.sum(mi)
    n = lax.fori_loop(0, E0_W // L, _filt, jnp.int32(0))

    z16i = jnp.zeros((L,), jnp.int32)
    t16i = jnp.full((L,), TRASH, jnp.int32)
    for t in range(K // L):
        srcflat[pl.ds(n + t * L, L)] = z16i
        dstflat[pl.ds(n + t * L, L)] = t16i

    nch = (n + (K - 1)) // K

    def _group(g, _):
        for b in range(G):
            j = g * G + b

            @pl.when(j < nch)
            def _():
                cb = j * K
                for k in range(K // L):
                    cs[b, pl.ds(k * L, L)] = srcflat[pl.ds(cb + k * L, L)]
                    cd[b, pl.ds(k * L, L)] = dstflat[pl.ds(cb + k * L, L)]
                pltpu.async_copy(x_hbm.at[cs.at[b]],
                                 rows.at[pl.ds(b * K, K)], sem)
        for b in range(G):
            j = g * G + b

            @pl.when(j < nch)
            def _():
                pltpu.make_async_copy(x_hbm.at[cs.at[b]],
                                      rows.at[pl.ds(b * K, K)], sem).wait()
        for b in range(G):
            j = g * G + b

            @pl.when(j < nch)
            def _():
                pltpu.async_copy(rows.at[pl.ds(b * K, K)],
                                 agg_sh.at[cd.at[b]], ssem, add=True)
        for b in range(G):
            j = g * G + b

            @pl.when(j < nch)
            def _():
                pltpu.make_async_copy(rows.at[pl.ds(b * K, K)],
                                      agg_sh.at[cd.at[b]], ssem).wait()
        return 0
    lax.fori_loop(0, (nch + G - 1) // G, _group, 0)

    base1 = wid * E1_W
    pltpu.sync_copy(dst1_hbm.at[pl.ds(base1, E1_W)], d1stage)

    def _c1(i, _):
        d16 = d1stage[pl.ds(i * L, L)]
        plsc.addupdate_scatter(cnt1loc, [d16], ones16)
        return 0
    lax.fori_loop(0, E1_W // L, _c1, 0)

    plsc.subcore_barrier()

    pltpu.sync_copy(agg_sh.at[pl.ds(sid * RPT, RPT)], rows.at[pl.ds(0, RPT)])
    pltpu.sync_copy(rows.at[pl.ds(0, RPT)],
                    s0p_hbm.at[cid, pl.ds(sid * RPT, RPT)])
    pltpu.sync_copy(cnt0loc, cnt0p_hbm.at[pl.ds(wid * R_ACC, R_ACC)])
    pltpu.sync_copy(cnt1loc, cnt1p_hbm.at[pl.ds(wid * N2, N2)])


_agg0_call = pl.kernel(
    _sc_agg0,
    out_type=[
        jax.ShapeDtypeStruct((NC, R_ACC, D_IN), jnp.float32),
        jax.ShapeDtypeStruct((NW * R_ACC,), jnp.float32),
        jax.ShapeDtypeStruct((NW * N2,), jnp.float32),
    ],
    mesh=_mesh,
    scratch_types=[
        pltpu.VMEM_SHARED((R_ACC, D_IN), jnp.float32),
        pltpu.VMEM((E0_W,), jnp.int32),
        pltpu.VMEM((E0_W,), jnp.int32),
        pltpu.VMEM((E0_W + 2 * K,), jnp.int32),
        pltpu.VMEM((E0_W + 2 * K,), jnp.int32),
        pltpu.VMEM((G, K), jnp.int32),
        pltpu.VMEM((G, K), jnp.int32),
        pltpu.VMEM((G * K, D_IN), jnp.float32),
        pltpu.VMEM((R_ACC,), jnp.float32),
        pltpu.VMEM((E1_W,), jnp.int32),
        pltpu.VMEM((N2,), jnp.float32),
        pltpu.SemaphoreType.DMA,
        pltpu.SemaphoreType.DMA,
    ],
    compiler_params=pltpu.CompilerParams(needs_layout_passes=False),
)


def _sc_agg1(h_hbm, src1_hbm, dst1_hbm, s1p_hbm, agg_sh, sstage, dstage, csr,
             cdr, rows, gsem, ssem):
    cid = lax.axis_index("c")
    sid = lax.axis_index("s")
    wid = sid * NC + cid

    z16f = jnp.zeros((L,), jnp.float32)

    def _zrow(i, _):
        for k in range(D_HID // L):
            rows[i, pl.ds(k * L, L)] = z16f
        return 0
    lax.fori_loop(0, RPT, _zrow, 0)
    pltpu.sync_copy(rows.at[pl.ds(0, RPT)], agg_sh.at[pl.ds(sid * RPT, RPT)])

    base = wid * E1_W
    pltpu.sync_copy(src1_hbm.at[pl.ds(base, E1_W)], sstage.at[pl.ds(0, E1_W)])
    pltpu.sync_copy(dst1_hbm.at[pl.ds(base, E1_W)], dstage.at[pl.ds(0, E1_W)])
    z16i = jnp.zeros((L,), jnp.int32)
    t16i = jnp.full((L,), TRASH, jnp.int32)
    for t in range((E1_PAD - E1_W) // L):
        sstage[pl.ds(E1_W + t * L, L)] = z16i
        dstage[pl.ds(E1_W + t * L, L)] = t16i

    for j in range(NCH1):
        for k in range(K // L):
            csr[j, pl.ds(k * L, L)] = sstage[pl.ds(j * K + k * L, L)]
            cdr[j, pl.ds(k * L, L)] = dstage[pl.ds(j * K + k * L, L)]
    gds = [pltpu.async_copy(h_hbm.at[csr.at[j]], rows.at[pl.ds(j * K, K)],
                            gsem) for j in range(NCH1)]
    plsc.subcore_barrier()
    for d in gds:
        d.wait()
    sds = [pltpu.async_copy(rows.at[pl.ds(j * K, K)], agg_sh.at[cdr.at[j]],
                            ssem, add=True) for j in range(NCH1)]
    for d in sds:
        d.wait()

    plsc.subcore_barrier()
    pltpu.sync_copy(agg_sh.at[pl.ds(sid * RPT, RPT)], rows.at[pl.ds(0, RPT)])
    pltpu.sync_copy(rows.at[pl.ds(0, RPT)],
                    s1p_hbm.at[cid, pl.ds(sid * RPT, RPT)])


_agg1_call = pl.kernel(
    _sc_agg1,
    out_type=jax.ShapeDtypeStruct((NC, R_ACC, D_HID), jnp.float32),
    mesh=_mesh,
    scratch_types=[
        pltpu.VMEM_SHARED((R_ACC, D_HID), jnp.float32),
        pltpu.VMEM((E1_PAD,), jnp.int32),
        pltpu.VMEM((E1_PAD,), jnp.int32),
        pltpu.VMEM((NCH1, K), jnp.int32),
        pltpu.VMEM((NCH1, K), jnp.int32),
        pltpu.VMEM((E1_PAD, D_HID), jnp.float32),
        pltpu.SemaphoreType.DMA,
        pltpu.SemaphoreType.DMA,
    ],
    compiler_params=pltpu.CompilerParams(needs_layout_passes=False),
)


def _tc_mid(s0p_ref, cnt0p_ref, x1_ref, w0l_ref, b0_ref, w0r_ref, w1r_ref,
            b1_ref, h_ref, r1_ref):
    s0 = s0p_ref[0, :N2, :] + s0p_ref[1, :N2, :]
    cnt = jnp.sum(cnt0p_ref[:, :N2], axis=0)
    agg = s0 / jnp.clip(cnt, 1.0)[:, None]
    h = (jnp.dot(agg, w0l_ref[...], preferred_element_type=jnp.float32)
         + b0_ref[0, :][None, :]
         + jnp.dot(x1_ref[...], w0r_ref[...],
                   preferred_element_type=jnp.float32))
    h = jnp.maximum(h, 0.0)
    h_ref[...] = h
    r1_ref[...] = (jnp.dot(h, w1r_ref[...], preferred_element_type=jnp.float32)
                   + b1_ref[0, :][None, :])


def _tc_out(s1p_ref, cnt1p_ref, r1_ref, w1l_ref, out_ref):
    s1 = s1p_ref[0, :N2, :] + s1p_ref[1, :N2, :]
    cnt = jnp.sum(cnt1p_ref[...], axis=0)
    agg = s1 / jnp.clip(cnt, 1.0)[:, None]
    z = (jnp.dot(agg, w1l_ref[...], preferred_element_type=jnp.float32)
         + r1_ref[...])
    m = jnp.max(z, axis=-1, keepdims=True)
    lse = jnp.log(jnp.sum(jnp.exp(z - m), axis=-1, keepdims=True)) + m
    out_ref[...] = z - lse


@jax.jit
def kernel(x, edge_index0, edge_index1, W0l, b0, W0r, W1l, b1, W1r):
    ei0 = edge_index0.astype(jnp.int32)
    ei1 = edge_index1.astype(jnp.int32)
    src0, dst0 = ei0[0], ei0[1]
    src1, dst1 = ei1[0], ei1[1]

    s0p, cnt0p, cnt1p = _agg0_call(x, src0, dst0, dst1)
    cnt0p = cnt0p.reshape(NW, R_ACC)
    cnt1p = cnt1p.reshape(NW, N2)

    h, r1 = pl.pallas_call(
        _tc_mid,
        out_shape=[
            jax.ShapeDtypeStruct((N2, D_HID), jnp.float32),
            jax.ShapeDtypeStruct((N2, D_OUT), jnp.float32),
        ],
    )(s0p, cnt0p, x[:N2], W0l, b0.reshape(1, -1), W0r, W1r,
      b1.reshape(1, -1))

    s1p = _agg1_call(h, src1, dst1)

    out = pl.pallas_call(
        _tc_out,
        out_shape=jax.ShapeDtypeStruct((N2, D_OUT), jnp.float32),
    )(s1p, cnt1p, r1, W1l)
    return out


# speedup vs baseline: 10.5461x; 1.4755x over previous
import jax
import jax.numpy as jnp
from jax import lax
from jax.experimental import pallas as pl
from jax.experimental.pallas import tpu as pltpu
from jax.experimental.pallas import tpu_sc as plsc

N0, N1, N2 = 50000, 10240, 1024
E0, E1 = 256000, 25600
D_IN, D_HID, D_OUT = 128, 128, 64

NC, NS, L = 2, 16, 16
NW = NC * NS
E0_W = E0 // NW
E1_W = E1 // NW
K = 128
TRASH = N2
R_ACC = 1152
RPT = R_ACC // NS
E1_PAD = (E1_W + K - 1) // K * K
NCH1 = E1_PAD // K
G = 2
CH1 = 3200
NCK1 = E1 // CH1
A1_ROWS = N2 // NW

_mesh = plsc.VectorSubcoreMesh(core_axis_name="c", subcore_axis_name="s")


def _sc_agg0(x_hbm, src0_hbm, dst0_hbm, src1_hbm, dst1_hbm, s0p_hbm,
             cnt0p_hbm, a1_hbm,
             agg_sh, srcstage, dststage, srcflat, dstflat, cs, cd, rows,
             cnt0loc, s1c, d1c, a1blk, sem, ssem):
    cid = lax.axis_index("c")
    sid = lax.axis_index("s")
    wid = sid * NC + cid

    z16f = jnp.zeros((L,), jnp.float32)
    ones16 = jnp.ones((L,), jnp.float32)

    def _zrow(i, _):
        for k in range(D_IN // L):
            rows[i, pl.ds(k * L, L)] = z16f
        return 0
    lax.fori_loop(0, RPT, _zrow, 0)

    def _zc0(i, _):
        cnt0loc[pl.ds(i * L, L)] = z16f
        return 0
    lax.fori_loop(0, R_ACC // L, _zc0, 0)

    def _za1(i, _):
        a1blk[pl.ds(i * L, L)] = z16f
        return 0
    lax.fori_loop(0, A1_ROWS * N2 // L, _za1, 0)

    pltpu.sync_copy(rows.at[pl.ds(0, RPT)], agg_sh.at[pl.ds(sid * RPT, RPT)])
    plsc.subcore_barrier()

    base = wid * E0_W
    pltpu.sync_copy(src0_hbm.at[pl.ds(base, E0_W)], srcstage)
    pltpu.sync_copy(dst0_hbm.at[pl.ds(base, E0_W)], dststage)

    def _filt(i, off):
        s16 = srcstage[pl.ds(i * L, L)]
        d16 = dststage[pl.ds(i * L, L)]
        mask = d16 < N2
        mi = mask.astype(jnp.int32)
        pos = off + plsc.cumsum(mi) - 1
        plsc.store_scatter(srcflat, [pos], s16, mask=mask)
        plsc.store_scatter(dstflat, [pos], d16, mask=mask)
        dcl = jnp.where(mask, d16, jnp.full((L,), TRASH, jnp.int32))
        plsc.addupdate_scatter(cnt0loc, [dcl], ones16)
        return off + jnp.sum(mi)
    n = lax.fori_loop(0, E0_W // L, _filt, jnp.int32(0))

    z16i = jnp.zeros((L,), jnp.int32)
    t16i = jnp.full((L,), TRASH, jnp.int32)
    for t in range(K // L):
        srcflat[pl.ds(n + t * L, L)] = z16i
        dstflat[pl.ds(n + t * L, L)] = t16i

    nch = (n + (K - 1)) // K

    def _group(g, _):
        for b in range(G):
            j = g * G + b

            @pl.when(j < nch)
            def _():
                cb = j * K
                for k in range(K // L):
                    cs[b, pl.ds(k * L, L)] = srcflat[pl.ds(cb + k * L, L)]
                    cd[b, pl.ds(k * L, L)] = dstflat[pl.ds(cb + k * L, L)]
                pltpu.async_copy(x_hbm.at[cs.at[b]],
                                 rows.at[pl.ds(b * K, K)], sem)
        for b in range(G):
            j = g * G + b

            @pl.when(j < nch)
            def _():
                pltpu.make_async_copy(x_hbm.at[cs.at[b]],
                                      rows.at[pl.ds(b * K, K)], sem).wait()
        for b in range(G):
            j = g * G + b

            @pl.when(j < nch)
            def _():
                pltpu.async_copy(rows.at[pl.ds(b * K, K)],
                                 agg_sh.at[cd.at[b]], ssem, add=True)
        for b in range(G):
            j = g * G + b

            @pl.when(j < nch)
            def _():
                pltpu.make_async_copy(rows.at[pl.ds(b * K, K)],
                                      agg_sh.at[cd.at[b]], ssem).wait()
        return 0
    lax.fori_loop(0, (nch + G - 1) // G, _group, 0)

    t32 = wid * A1_ROWS
    z16i2 = jnp.zeros((L,), jnp.int32)
    for c in range(NCK1):
        pltpu.sync_copy(src1_hbm.at[pl.ds(c * CH1, CH1)], s1c)
        pltpu.sync_copy(dst1_hbm.at[pl.ds(c * CH1, CH1)], d1c)

        def _a1scan(i, _):
            s16 = s1c[pl.ds(i * L, L)]
            d16 = d1c[pl.ds(i * L, L)]
            dloc = d16 - t32
            msk = jnp.logical_and(dloc >= 0, dloc < A1_ROWS)
            flat = jnp.where(msk, dloc * N2 + s16, z16i2)
            plsc.addupdate_scatter(a1blk, [flat], ones16, mask=msk)
            return 0
        lax.fori_loop(0, CH1 // L, _a1scan, 0)

    plsc.subcore_barrier()

    pltpu.sync_copy(agg_sh.at[pl.ds(sid * RPT, RPT)], rows.at[pl.ds(0, RPT)])
    pltpu.sync_copy(rows.at[pl.ds(0, RPT)],
                    s0p_hbm.at[cid, pl.ds(sid * RPT, RPT)])
    pltpu.sync_copy(cnt0loc, cnt0p_hbm.at[pl.ds(wid * R_ACC, R_ACC)])
    pltpu.sync_copy(a1blk, a1_hbm.at[pl.ds(wid * A1_ROWS * N2, A1_ROWS * N2)])


_agg0_call = pl.kernel(
    _sc_agg0,
    out_type=[
        jax.ShapeDtypeStruct((NC, R_ACC, D_IN), jnp.float32),
        jax.ShapeDtypeStruct((NW * R_ACC,), jnp.float32),
        jax.ShapeDtypeStruct((N2 * N2,), jnp.float32),
    ],
    mesh=_mesh,
    scratch_types=[
        pltpu.VMEM_SHARED((R_ACC, D_IN), jnp.float32),
        pltpu.VMEM((E0_W,), jnp.int32),
        pltpu.VMEM((E0_W,), jnp.int32),
        pltpu.VMEM((E0_W + 2 * K,), jnp.int32),
        pltpu.VMEM((E0_W + 2 * K,), jnp.int32),
        pltpu.VMEM((G, K), jnp.int32),
        pltpu.VMEM((G, K), jnp.int32),
        pltpu.VMEM((G * K, D_IN), jnp.float32),
        pltpu.VMEM((R_ACC,), jnp.float32),
        pltpu.VMEM((CH1,), jnp.int32),
        pltpu.VMEM((CH1,), jnp.int32),
        pltpu.VMEM((A1_ROWS * N2,), jnp.float32),
        pltpu.SemaphoreType.DMA,
        pltpu.SemaphoreType.DMA,
    ],
    compiler_params=pltpu.CompilerParams(needs_layout_passes=False),
)


def _tc_fused(s0p_ref, cnt0p_ref, a1_ref, x1_ref, w0l_ref, b0_ref, w0r_ref,
              w1l_ref, b1_ref, w1r_ref, out_ref):
    s0 = s0p_ref[0, :N2, :] + s0p_ref[1, :N2, :]
    cnt0 = jnp.sum(cnt0p_ref[:, :N2], axis=0)
    agg0 = s0 / jnp.clip(cnt0, 1.0)[:, None]
    h = (jnp.dot(agg0, w0l_ref[...], preferred_element_type=jnp.float32)
         + b0_ref[0, :][None, :]
         + jnp.dot(x1_ref[...], w0r_ref[...],
                   preferred_element_type=jnp.float32))
    h = jnp.maximum(h, 0.0)
    a1 = a1_ref[...]
    s1 = jnp.dot(a1, h, preferred_element_type=jnp.float32)
    cnt1 = jnp.sum(a1, axis=1)
    agg1 = s1 / jnp.clip(cnt1, 1.0)[:, None]
    z = (jnp.dot(agg1, w1l_ref[...], preferred_element_type=jnp.float32)
         + b1_ref[0, :][None, :]
         + jnp.dot(h, w1r_ref[...], preferred_element_type=jnp.float32))
    m = jnp.max(z, axis=-1, keepdims=True)
    lse = jnp.log(jnp.sum(jnp.exp(z - m), axis=-1, keepdims=True)) + m
    out_ref[...] = z - lse


@jax.jit
def kernel(x, edge_index0, edge_index1, W0l, b0, W0r, W1l, b1, W1r):
    ei0 = edge_index0.astype(jnp.int32)
    ei1 = edge_index1.astype(jnp.int32)
    src0, dst0 = ei0[0], ei0[1]
    src1, dst1 = ei1[0], ei1[1]

    s0p, cnt0p, a1 = _agg0_call(x, src0, dst0, src1, dst1)
    cnt0p = cnt0p.reshape(NW, R_ACC)
    a1 = a1.reshape(N2, N2)

    out = pl.pallas_call(
        _tc_fused,
        out_shape=jax.ShapeDtypeStruct((N2, D_OUT), jnp.float32),
    )(s0p, cnt0p, a1, x[:N2], W0l, b0.reshape(1, -1), W0r, W1l,
      b1.reshape(1, -1), W1r)
    return out


# speedup vs baseline: 17.4623x; 1.6558x over previous
import jax
import jax.numpy as jnp
from jax import lax
from jax.experimental import pallas as pl
from jax.experimental.pallas import tpu as pltpu
from jax.experimental.pallas import tpu_sc as plsc

N0, N1, N2 = 50000, 10240, 1024
E0, E1 = 256000, 25600
D_IN, D_HID, D_OUT = 128, 128, 64

NC, NS, L = 2, 16, 16
NW = NC * NS
E0_W = E0 // NW
E1_W = E1 // NW
K = 128
TRASH = N2
R_ACC = 1152
RPT = R_ACC // NS
E1_PAD = (E1_W + K - 1) // K * K
NCH1 = E1_PAD // K
G = 2
CH1 = 3200
NCK1 = E1 // CH1
A1_ROWS = N2 // NW

_mesh = plsc.VectorSubcoreMesh(core_axis_name="c", subcore_axis_name="s")


def _sc_agg0(x_hbm, src0_hbm, dst0_hbm, s0p_hbm,
             agg_sh, xtab_sh, srcstage, dststage, srcflo, dstflo, srcfhi,
             dstfhi, cs, cd, rows, sem, ssem):
    cid = lax.axis_index("c")
    sid = lax.axis_index("s")
    wid = sid * NC + cid

    z16f = jnp.zeros((L,), jnp.float32)
    ones16 = jnp.ones((L,), jnp.float32)
    XT = N1 // 2
    XPT = XT // NS

    def _zrow(i, _):
        for k in range(D_IN // L):
            rows[i, pl.ds(k * L, L)] = z16f
        return 0
    lax.fori_loop(0, RPT, _zrow, 0)

    pltpu.sync_copy(rows.at[pl.ds(0, RPT)], agg_sh.at[pl.ds(sid * RPT, RPT)])
    pltpu.sync_copy(x_hbm.at[pl.ds(sid * XPT, XPT)],
                    xtab_sh.at[pl.ds(sid * XPT, XPT)])

    base = wid * E0_W
    EST = E0_W // 2
    xtv = jnp.full((L,), XT, jnp.int32)

    def _filt(i, carry):
        off_lo, off_hi = carry
        s16 = srcstage[pl.ds(i * L, L)]
        d16 = dststage[pl.ds(i * L, L)]
        keep = d16 < N2
        lo = jnp.logical_and(keep, s16 < XT)
        hi = jnp.logical_and(keep, s16 >= XT)
        mlo = lo.astype(jnp.int32)
        mhi = hi.astype(jnp.int32)
        pos_lo = off_lo + plsc.cumsum(mlo) - 1
        pos_hi = off_hi + plsc.cumsum(mhi) - 1
        plsc.store_scatter(srcflo, [pos_lo], s16, mask=lo)
        plsc.store_scatter(dstflo, [pos_lo], d16, mask=lo)
        plsc.store_scatter(srcfhi, [pos_hi], s16 - xtv, mask=hi)
        plsc.store_scatter(dstfhi, [pos_hi], d16, mask=hi)
        return (off_lo + jnp.sum(mlo), off_hi + jnp.sum(mhi))
    fc = (jnp.int32(0), jnp.int32(0))
    for r in range(2):
        pltpu.sync_copy(src0_hbm.at[pl.ds(base + r * EST, EST)], srcstage)
        pltpu.sync_copy(dst0_hbm.at[pl.ds(base + r * EST, EST)], dststage)
        fc = lax.fori_loop(0, EST // L, _filt, fc)
    n_lo, n_hi = fc

    z16i = jnp.zeros((L,), jnp.int32)
    t16i = jnp.full((L,), TRASH, jnp.int32)
    for t in range(K // L):
        srcflo[pl.ds(n_lo + t * L, L)] = z16i
        dstflo[pl.ds(n_lo + t * L, L)] = t16i
        srcfhi[pl.ds(n_hi + t * L, L)] = z16i
        dstfhi[pl.ds(n_hi + t * L, L)] = t16i

    plsc.subcore_barrier()

    def _run_half(srcf, dstf, n_h):
        nch = (n_h + (K - 1)) // K

        def _group(g, _):
            for b in range(G):
                j = g * G + b

                @pl.when(j < nch)
                def _():
                    cb = j * K
                    for k in range(K // L):
                        cs[b, pl.ds(k * L, L)] = srcf[pl.ds(cb + k * L, L)]
                        cd[b, pl.ds(k * L, L)] = dstf[pl.ds(cb + k * L, L)]
                    pltpu.async_copy(xtab_sh.at[cs.at[b]],
                                     rows.at[pl.ds(b * K, K)], sem)
            for b in range(G):
                j = g * G + b

                @pl.when(j < nch)
                def _():
                    pltpu.make_async_copy(
                        xtab_sh.at[cs.at[b]],
                        rows.at[pl.ds(b * K, K)], sem).wait()
            for b in range(G):
                j = g * G + b

                @pl.when(j < nch)
                def _():
                    pltpu.async_copy(rows.at[pl.ds(b * K, K)],
                                     agg_sh.at[cd.at[b]], ssem, add=True)
            for b in range(G):
                j = g * G + b

                @pl.when(j < nch)
                def _():
                    pltpu.make_async_copy(rows.at[pl.ds(b * K, K)],
                                          agg_sh.at[cd.at[b]], ssem).wait()
            return 0
        lax.fori_loop(0, (nch + G - 1) // G, _group, 0)

    _run_half(srcflo, dstflo, n_lo)
    plsc.subcore_barrier()
    pltpu.sync_copy(x_hbm.at[pl.ds(XT + sid * XPT, XPT)],
                    xtab_sh.at[pl.ds(sid * XPT, XPT)])
    plsc.subcore_barrier()
    _run_half(srcfhi, dstfhi, n_hi)

    plsc.subcore_barrier()

    pltpu.sync_copy(agg_sh.at[pl.ds(sid * RPT, RPT)], rows.at[pl.ds(0, RPT)])
    pltpu.sync_copy(rows.at[pl.ds(0, RPT)],
                    s0p_hbm.at[cid, pl.ds(sid * RPT, RPT)])


_agg0_call = pl.kernel(
    _sc_agg0,
    out_type=jax.ShapeDtypeStruct((NC, R_ACC, D_IN), jnp.float32),
    mesh=_mesh,
    scratch_types=[
        pltpu.VMEM_SHARED((R_ACC, D_IN), jnp.float32),
        pltpu.VMEM_SHARED((N1 // 2, D_IN), jnp.float32),
        pltpu.VMEM((E0_W // 2,), jnp.int32),
        pltpu.VMEM((E0_W // 2,), jnp.int32),
        pltpu.VMEM((E0_W + 2 * K,), jnp.int32),
        pltpu.VMEM((E0_W + 2 * K,), jnp.int32),
        pltpu.VMEM((E0_W + 2 * K,), jnp.int32),
        pltpu.VMEM((E0_W + 2 * K,), jnp.int32),
        pltpu.VMEM((G, K), jnp.int32),
        pltpu.VMEM((G, K), jnp.int32),
        pltpu.VMEM((G * K, D_IN), jnp.float32),
        pltpu.SemaphoreType.DMA,
        pltpu.SemaphoreType.DMA,
    ],
    compiler_params=pltpu.CompilerParams(needs_layout_passes=False),
)


def _sc_a1(src1_hbm, dst1_hbm, dst0_hbm, a1_hbm, cnt0p_hbm, s1c, d1c, a1blk,
           d0stage, cnt0loc):
    cid = lax.axis_index("c")
    sid = lax.axis_index("s")
    wid = sid * NC + cid

    z16f = jnp.zeros((L,), jnp.float32)
    ones16 = jnp.ones((L,), jnp.float32)

    def _za1(i, _):
        a1blk[pl.ds(i * L, L)] = z16f
        return 0
    lax.fori_loop(0, A1_ROWS * N2 // L, _za1, 0)

    def _zc0(i, _):
        cnt0loc[pl.ds(i * L, L)] = z16f
        return 0
    lax.fori_loop(0, R_ACC // L, _zc0, 0)

    pltpu.sync_copy(dst0_hbm.at[pl.ds(wid * E0_W, E0_W)], d0stage)
    t16c = jnp.full((L,), TRASH, jnp.int32)

    def _c0(i, _):
        d16 = d0stage[pl.ds(i * L, L)]
        keep = d16 < N2
        dcl = jnp.where(keep, d16, t16c)
        plsc.addupdate_scatter(cnt0loc, [dcl], ones16)
        return 0
    lax.fori_loop(0, E0_W // L, _c0, 0)
    pltpu.sync_copy(cnt0loc, cnt0p_hbm.at[pl.ds(wid * R_ACC, R_ACC)])

    t32 = wid * A1_ROWS
    z16i2 = jnp.zeros((L,), jnp.int32)
    for c in range(NCK1):
        pltpu.sync_copy(src1_hbm.at[pl.ds(c * CH1, CH1)], s1c)
        pltpu.sync_copy(dst1_hbm.at[pl.ds(c * CH1, CH1)], d1c)

        def _a1scan(i, _):
            s16 = s1c[pl.ds(i * L, L)]
            d16 = d1c[pl.ds(i * L, L)]
            dloc = d16 - t32
            msk = jnp.logical_and(dloc >= 0, dloc < A1_ROWS)
            flat = jnp.where(msk, dloc * N2 + s16, z16i2)
            plsc.addupdate_scatter(a1blk, [flat], ones16, mask=msk)
            return 0
        lax.fori_loop(0, CH1 // L, _a1scan, 0)

    pltpu.sync_copy(a1blk, a1_hbm.at[pl.ds(wid * A1_ROWS * N2, A1_ROWS * N2)])


_a1_call = pl.kernel(
    _sc_a1,
    out_type=[
        jax.ShapeDtypeStruct((N2 * N2,), jnp.float32),
        jax.ShapeDtypeStruct((NW * R_ACC,), jnp.float32),
    ],
    mesh=_mesh,
    scratch_types=[
        pltpu.VMEM((CH1,), jnp.int32),
        pltpu.VMEM((CH1,), jnp.int32),
        pltpu.VMEM((A1_ROWS * N2,), jnp.float32),
        pltpu.VMEM((E0_W,), jnp.int32),
        pltpu.VMEM((R_ACC,), jnp.float32),
    ],
    compiler_params=pltpu.CompilerParams(needs_layout_passes=False),
)


def _tc_fused(s0p_ref, cnt0p_ref, a1_ref, x1_ref, w0l_ref, b0_ref, w0r_ref,
              w1l_ref, b1_ref, w1r_ref, out_ref):
    s0 = s0p_ref[0, :N2, :] + s0p_ref[1, :N2, :]
    cnt0 = jnp.sum(cnt0p_ref[:, :N2], axis=0)
    agg0 = s0 / jnp.clip(cnt0, 1.0)[:, None]
    h = (jnp.dot(agg0, w0l_ref[...], preferred_element_type=jnp.float32)
         + b0_ref[0, :][None, :]
         + jnp.dot(x1_ref[...], w0r_ref[...],
                   preferred_element_type=jnp.float32))
    h = jnp.maximum(h, 0.0)
    a1 = a1_ref[...]
    s1 = jnp.dot(a1, h, preferred_element_type=jnp.float32)
    cnt1 = jnp.sum(a1, axis=1)
    agg1 = s1 / jnp.clip(cnt1, 1.0)[:, None]
    z = (jnp.dot(agg1, w1l_ref[...], preferred_element_type=jnp.float32)
         + b1_ref[0, :][None, :]
         + jnp.dot(h, w1r_ref[...], preferred_element_type=jnp.float32))
    m = jnp.max(z, axis=-1, keepdims=True)
    lse = jnp.log(jnp.sum(jnp.exp(z - m), axis=-1, keepdims=True)) + m
    out_ref[...] = z - lse


@jax.jit
def kernel(x, edge_index0, edge_index1, W0l, b0, W0r, W1l, b1, W1r):
    ei0 = edge_index0.astype(jnp.int32)
    ei1 = edge_index1.astype(jnp.int32)
    src0, dst0 = ei0[0], ei0[1]
    src1, dst1 = ei1[0], ei1[1]

    s0p = _agg0_call(x, src0, dst0)
    a1, cnt0p = _a1_call(src1, dst1, dst0)
    cnt0p = cnt0p.reshape(NW, R_ACC)
    a1 = a1.reshape(N2, N2)

    out = pl.pallas_call(
        _tc_fused,
        out_shape=jax.ShapeDtypeStruct((N2, D_OUT), jnp.float32),
    )(s0p, cnt0p, a1, x[:N2], W0l, b0.reshape(1, -1), W0r, W1l,
      b1.reshape(1, -1), W1r)
    return out


# speedup vs baseline: 20.8444x; 1.1937x over previous
import jax
import jax.numpy as jnp
from jax import lax
from jax.experimental import pallas as pl
from jax.experimental.pallas import tpu as pltpu
from jax.experimental.pallas import tpu_sc as plsc

N0, N1, N2 = 50000, 10240, 1024
E0, E1 = 256000, 25600
D_IN, D_HID, D_OUT = 128, 128, 64

NC, NS, L = 2, 16, 16
NW = NC * NS
E0_W = E0 // NW
E1_W = E1 // NW
K = 128
TRASH = N2
R_ACC = 1152
RPT = R_ACC // NS
E1_PAD = (E1_W + K - 1) // K * K
NCH1 = E1_PAD // K
G = 2
CH1 = 3200
NCK1 = E1 // CH1
A1_ROWS = N2 // NW

_mesh = plsc.VectorSubcoreMesh(core_axis_name="c", subcore_axis_name="s")


def _sc_agg0(x_hbm, src0_hbm, dst0_hbm, s0p_hbm,
             agg_sh, xtab_sh, srcstage, dststage, srcflo, dstflo, srcfhi,
             dstfhi, cs, cd, rows, sem, ssem):
    cid = lax.axis_index("c")
    sid = lax.axis_index("s")
    wid = sid * NC + cid

    z16f = jnp.zeros((L,), jnp.float32)
    ones16 = jnp.ones((L,), jnp.float32)
    XT = N1 // 2
    XPT = XT // NS

    def _zrow(i, _):
        for k in range(D_IN // L):
            rows[i, pl.ds(k * L, L)] = z16f
        return 0
    lax.fori_loop(0, RPT, _zrow, 0)

    pltpu.sync_copy(rows.at[pl.ds(0, RPT)], agg_sh.at[pl.ds(sid * RPT, RPT)])
    pltpu.sync_copy(x_hbm.at[pl.ds(sid * XPT, XPT)],
                    xtab_sh.at[pl.ds(sid * XPT, XPT)])

    base = wid * E0_W
    EST = E0_W // 2
    xtv = jnp.full((L,), XT, jnp.int32)

    def _filt(i, carry):
        off_lo, off_hi = carry
        s16 = srcstage[pl.ds(i * L, L)]
        d16 = dststage[pl.ds(i * L, L)]
        keep = d16 < N2
        lo = jnp.logical_and(keep, s16 < XT)
        hi = jnp.logical_and(keep, s16 >= XT)
        mlo = lo.astype(jnp.int32)
        mhi = hi.astype(jnp.int32)
        pos_lo = off_lo + plsc.cumsum(mlo) - 1
        pos_hi = off_hi + plsc.cumsum(mhi) - 1
        plsc.store_scatter(srcflo, [pos_lo], s16, mask=lo)
        plsc.store_scatter(dstflo, [pos_lo], d16, mask=lo)
        plsc.store_scatter(srcfhi, [pos_hi], s16 - xtv, mask=hi)
        plsc.store_scatter(dstfhi, [pos_hi], d16, mask=hi)
        return (off_lo + jnp.sum(mlo), off_hi + jnp.sum(mhi))
    fc = (jnp.int32(0), jnp.int32(0))
    for r in range(2):
        pltpu.sync_copy(src0_hbm.at[pl.ds(base + r * EST, EST)], srcstage)
        pltpu.sync_copy(dst0_hbm.at[pl.ds(base + r * EST, EST)], dststage)
        fc = lax.fori_loop(0, EST // L, _filt, fc)
    n_lo, n_hi = fc

    z16i = jnp.zeros((L,), jnp.int32)
    t16i = jnp.full((L,), TRASH, jnp.int32)
    for t in range(K // L):
        srcflo[pl.ds(n_lo + t * L, L)] = z16i
        dstflo[pl.ds(n_lo + t * L, L)] = t16i
        srcfhi[pl.ds(n_hi + t * L, L)] = z16i
        dstfhi[pl.ds(n_hi + t * L, L)] = t16i

    plsc.subcore_barrier()

    def _run_half(srcf, dstf, n_h):
        nch = (n_h + (K - 1)) // K

        def _group(g, _):
            for b in range(G):
                j = g * G + b

                @pl.when(j < nch)
                def _():
                    cb = j * K
                    for k in range(K // L):
                        cs[b, pl.ds(k * L, L)] = srcf[pl.ds(cb + k * L, L)]
                        cd[b, pl.ds(k * L, L)] = dstf[pl.ds(cb + k * L, L)]
                    pltpu.async_copy(xtab_sh.at[cs.at[b]],
                                     rows.at[pl.ds(b * K, K)], sem)
            for b in range(G):
                j = g * G + b

                @pl.when(j < nch)
                def _():
                    pltpu.make_async_copy(
                        xtab_sh.at[cs.at[b]],
                        rows.at[pl.ds(b * K, K)], sem).wait()
            for b in range(G):
                j = g * G + b

                @pl.when(j < nch)
                def _():
                    pltpu.async_copy(rows.at[pl.ds(b * K, K)],
                                     agg_sh.at[cd.at[b]], ssem, add=True)
            for b in range(G):
                j = g * G + b

                @pl.when(j < nch)
                def _():
                    pltpu.make_async_copy(rows.at[pl.ds(b * K, K)],
                                          agg_sh.at[cd.at[b]], ssem).wait()
            return 0
        lax.fori_loop(0, (nch + G - 1) // G, _group, 0)

    _run_half(srcflo, dstflo, n_lo)
    plsc.subcore_barrier()
    pltpu.sync_copy(x_hbm.at[pl.ds(XT + sid * XPT, XPT)],
                    xtab_sh.at[pl.ds(sid * XPT, XPT)])
    plsc.subcore_barrier()
    _run_half(srcfhi, dstfhi, n_hi)

    plsc.subcore_barrier()

    pltpu.sync_copy(agg_sh.at[pl.ds(sid * RPT, RPT)], rows.at[pl.ds(0, RPT)])
    pltpu.sync_copy(rows.at[pl.ds(0, RPT)],
                    s0p_hbm.at[cid, pl.ds(sid * RPT, RPT)])


_agg0_call = pl.kernel(
    _sc_agg0,
    out_type=jax.ShapeDtypeStruct((NC, R_ACC, D_IN), jnp.float32),
    mesh=_mesh,
    scratch_types=[
        pltpu.VMEM_SHARED((R_ACC, D_IN), jnp.float32),
        pltpu.VMEM_SHARED((N1 // 2, D_IN), jnp.float32),
        pltpu.VMEM((E0_W // 2,), jnp.int32),
        pltpu.VMEM((E0_W // 2,), jnp.int32),
        pltpu.VMEM((E0_W + 2 * K,), jnp.int32),
        pltpu.VMEM((E0_W + 2 * K,), jnp.int32),
        pltpu.VMEM((E0_W + 2 * K,), jnp.int32),
        pltpu.VMEM((E0_W + 2 * K,), jnp.int32),
        pltpu.VMEM((G, K), jnp.int32),
        pltpu.VMEM((G, K), jnp.int32),
        pltpu.VMEM((G * K, D_IN), jnp.float32),
        pltpu.SemaphoreType.DMA,
        pltpu.SemaphoreType.DMA,
    ],
    compiler_params=pltpu.CompilerParams(needs_layout_passes=False),
)


def _sc_a1(src1_hbm, dst1_hbm, dst0_hbm, a1_hbm, cnt0p_hbm, s1c, d1c, a1blk,
           d0stage, cnt0loc, esem0, esem1, dsem):
    cid = lax.axis_index("c")
    sid = lax.axis_index("s")
    wid = sid * NC + cid

    z16f = jnp.zeros((L,), jnp.float32)
    ones16 = jnp.ones((L,), jnp.float32)

    ds0 = pltpu.async_copy(dst0_hbm.at[pl.ds(wid * E0_W, E0_W)], d0stage,
                           dsem)
    bsem = [esem0, esem1]
    pre = [(pltpu.async_copy(src1_hbm.at[pl.ds(0, CH1)], s1c.at[0], esem0),
            pltpu.async_copy(dst1_hbm.at[pl.ds(0, CH1)], d1c.at[0], esem0))]

    def _za1(i, _):
        for u in range(8):
            a1blk[pl.ds(i * 8 * L + u * L, L)] = z16f
        return 0
    lax.fori_loop(0, A1_ROWS * N2 // L // 8, _za1, 0)

    def _zc0(i, _):
        cnt0loc[pl.ds(i * L, L)] = z16f
        return 0
    lax.fori_loop(0, R_ACC // L, _zc0, 0)

    t32 = wid * A1_ROWS
    z16i2 = jnp.zeros((L,), jnp.int32)
    for c in range(NCK1):
        b = c % 2
        if c + 1 < NCK1:
            nb = (c + 1) % 2
            pre.append(
                (pltpu.async_copy(src1_hbm.at[pl.ds((c + 1) * CH1, CH1)],
                                  s1c.at[nb], bsem[nb]),
                 pltpu.async_copy(dst1_hbm.at[pl.ds((c + 1) * CH1, CH1)],
                                  d1c.at[nb], bsem[nb])))
        pre[c][0].wait()
        pre[c][1].wait()

        def _a1scan(i, _):
            for u in range(2):
                s16 = s1c[b, pl.ds(i * 2 * L + u * L, L)]
                d16 = d1c[b, pl.ds(i * 2 * L + u * L, L)]
                dloc = d16 - t32
                msk = jnp.logical_and(dloc >= 0, dloc < A1_ROWS)
                flat = jnp.where(msk, dloc * N2 + s16, z16i2)
                plsc.addupdate_scatter(a1blk, [flat], ones16, mask=msk)
            return 0
        lax.fori_loop(0, CH1 // L // 2, _a1scan, 0)

    pltpu.sync_copy(a1blk, a1_hbm.at[pl.ds(wid * A1_ROWS * N2, A1_ROWS * N2)])

    ds0.wait()
    t16c = jnp.full((L,), TRASH, jnp.int32)

    def _c0(i, _):
        for u in range(2):
            d16 = d0stage[pl.ds(i * 2 * L + u * L, L)]
            keep = d16 < N2
            dcl = jnp.where(keep, d16, t16c)
            plsc.addupdate_scatter(cnt0loc, [dcl], ones16)
        return 0
    lax.fori_loop(0, E0_W // L // 2, _c0, 0)
    pltpu.sync_copy(cnt0loc, cnt0p_hbm.at[pl.ds(wid * R_ACC, R_ACC)])


_a1_call = pl.kernel(
    _sc_a1,
    out_type=[
        jax.ShapeDtypeStruct((N2 * N2,), jnp.float32),
        jax.ShapeDtypeStruct((NW * R_ACC,), jnp.float32),
    ],
    mesh=_mesh,
    scratch_types=[
        pltpu.VMEM((2, CH1), jnp.int32),
        pltpu.VMEM((2, CH1), jnp.int32),
        pltpu.VMEM((A1_ROWS * N2,), jnp.float32),
        pltpu.VMEM((E0_W,), jnp.int32),
        pltpu.VMEM((R_ACC,), jnp.float32),
        pltpu.SemaphoreType.DMA,
        pltpu.SemaphoreType.DMA,
        pltpu.SemaphoreType.DMA,
    ],
    compiler_params=pltpu.CompilerParams(needs_layout_passes=False),
)


def _tc_fused(s0p_ref, cnt0p_ref, a1_ref, x1_ref, w0l_ref, b0_ref, w0r_ref,
              w1l_ref, b1_ref, w1r_ref, out_ref):
    s0 = s0p_ref[0, :N2, :] + s0p_ref[1, :N2, :]
    cnt0 = jnp.sum(cnt0p_ref[:, :N2], axis=0)
    agg0 = s0 / jnp.clip(cnt0, 1.0)[:, None]
    h = (jnp.dot(agg0, w0l_ref[...], preferred_element_type=jnp.float32)
         + b0_ref[0, :][None, :]
         + jnp.dot(x1_ref[...], w0r_ref[...],
                   preferred_element_type=jnp.float32))
    h = jnp.maximum(h, 0.0)
    a1 = a1_ref[...]
    s1 = jnp.dot(a1, h, preferred_element_type=jnp.float32)
    cnt1 = jnp.sum(a1, axis=1)
    agg1 = s1 / jnp.clip(cnt1, 1.0)[:, None]
    z = (jnp.dot(agg1, w1l_ref[...], preferred_element_type=jnp.float32)
         + b1_ref[0, :][None, :]
         + jnp.dot(h, w1r_ref[...], preferred_element_type=jnp.float32))
    m = jnp.max(z, axis=-1, keepdims=True)
    lse = jnp.log(jnp.sum(jnp.exp(z - m), axis=-1, keepdims=True)) + m
    out_ref[...] = z - lse


@jax.jit
def kernel(x, edge_index0, edge_index1, W0l, b0, W0r, W1l, b1, W1r):
    ei0 = edge_index0.astype(jnp.int32)
    ei1 = edge_index1.astype(jnp.int32)
    src0, dst0 = ei0[0], ei0[1]
    src1, dst1 = ei1[0], ei1[1]

    s0p = _agg0_call(x, src0, dst0)
    a1, cnt0p = _a1_call(src1, dst1, dst0)
    cnt0p = cnt0p.reshape(NW, R_ACC)
    a1 = a1.reshape(N2, N2)

    out = pl.pallas_call(
        _tc_fused,
        out_shape=jax.ShapeDtypeStruct((N2, D_OUT), jnp.float32),
    )(s0p, cnt0p, a1, x[:N2], W0l, b0.reshape(1, -1), W0r, W1l,
      b1.reshape(1, -1), W1r)
    return out
